# Initial kernel scaffold; baseline (speedup 1.0000x reference)
import jax
import jax.numpy as jnp
from jax.experimental import pallas as pl


def kernel(x, edge_index, edge_weight, W, b):
    raise NotImplementedError("write your pallas kernel here")


# speedup vs baseline: 4.4907x; 4.4907x over previous
import functools

import jax
import jax.numpy as jnp
from jax import lax
from jax.experimental import pallas as pl
from jax.experimental.pallas import tpu as pltpu
from jax.experimental.pallas import tpu_sc as plsc

N_NODES = 10000
N_EDGES = 320000
D = 128

NC = 2
NS = 16
L = 16
NW = NC * NS
EPW = N_EDGES // NW
CHUNK = 80
NCHUNK = EPW // CHUNK
N_PAD = 10240
RPT = N_PAD // NS
ZROWS = 64


def _mm_body(x_ref, w_ref, o_ref):
    o_ref[...] = jnp.dot(x_ref[...], w_ref[...], preferred_element_type=jnp.float32)


def _matmul(x, W):
    BM = 2000
    return pl.pallas_call(
        _mm_body,
        grid=(N_NODES // BM,),
        in_specs=[
            pl.BlockSpec((BM, D), lambda i: (i, 0)),
            pl.BlockSpec((D, D), lambda i: (0, 0)),
        ],
        out_specs=pl.BlockSpec((BM, D), lambda i: (i, 0)),
        out_shape=jax.ShapeDtypeStruct((N_NODES, D), jnp.float32),
    )(x, W)


def _sc_body(h_hbm, row_hbm, col_hbm, w_hbm, part_hbm,
             colv, rowv, wv, rows_v, zer_v, accum, sem):
    cid = lax.axis_index("c")
    sid = lax.axis_index("s")
    wid = sid * NC + cid

    def z1(i, _):
        r = i // (D // L)
        g = i % (D // L)
        zer_v[r, pl.ds(g * L, L)] = jnp.zeros((L,), jnp.float32)
        return 0
    lax.fori_loop(0, ZROWS * (D // L), z1, 0)

    base_rows = sid * RPT

    def z2(k, _):
        pltpu.sync_copy(zer_v, accum.at[pl.ds(base_rows + k * ZROWS, ZROWS)])
        return 0
    lax.fori_loop(0, RPT // ZROWS, z2, 0)
    plsc.subcore_barrier()

    ebase = wid * EPW

    def chunk_body(i, _):
        off = ebase + i * CHUNK
        pltpu.sync_copy(col_hbm.at[pl.ds(off, CHUNK)], colv)
        pltpu.sync_copy(row_hbm.at[pl.ds(off, CHUNK)], rowv)
        pltpu.sync_copy(w_hbm.at[pl.ds(off, CHUNK)], wv)
        pltpu.async_copy(h_hbm.at[colv], rows_v, sem).wait()

        def scale(g, _):
            wg = wv[pl.ds(g * L, L)]
            for l in range(L):
                e = g * L + l
                wb = jnp.full((L,), wg[l], jnp.float32)
                for d in range(D // L):
                    sl = pl.ds(d * L, L)
                    rows_v[e, sl] = rows_v[e, sl] * wb
            return 0
        lax.fori_loop(0, CHUNK // L, scale, 0)

        pltpu.sync_copy(rows_v, accum.at[rowv], add=True)
        return 0
    lax.fori_loop(0, NCHUNK, chunk_body, 0)
    plsc.subcore_barrier()

    def wout(k, _):
        r0 = base_rows + k * ZROWS
        pltpu.sync_copy(accum.at[pl.ds(r0, ZROWS)], part_hbm.at[cid, pl.ds(r0, ZROWS)])
        return 0
    lax.fori_loop(0, RPT // ZROWS, wout, 0)


def _sc_spmm(h, row, col, w):
    mesh = plsc.VectorSubcoreMesh(
        core_axis_name="c", subcore_axis_name="s", num_cores=NC, num_subcores=NS)
    f = pl.kernel(
        _sc_body,
        out_type=jax.ShapeDtypeStruct((NC, N_PAD, D), jnp.float32),
        mesh=mesh,
        scratch_types=[
            pltpu.VMEM((CHUNK,), jnp.int32),
            pltpu.VMEM((CHUNK,), jnp.int32),
            pltpu.VMEM((CHUNK,), jnp.float32),
            pltpu.VMEM((CHUNK, D), jnp.float32),
            pltpu.VMEM((ZROWS, D), jnp.float32),
            pltpu.VMEM_SHARED((N_PAD, D), jnp.float32),
            pltpu.SemaphoreType.DMA,
        ],
    )
    return f(h, row, col, w)


def _comb_body(p_ref, b_ref, o_ref):
    o_ref[...] = p_ref[0] + p_ref[1] + b_ref[...]


def _combine(parts, b):
    BM = 2000
    return pl.pallas_call(
        _comb_body,
        grid=(N_NODES // BM,),
        in_specs=[
            pl.BlockSpec((NC, BM, D), lambda i: (0, i, 0)),
            pl.BlockSpec((1, D), lambda i: (0, 0)),
        ],
        out_specs=pl.BlockSpec((BM, D), lambda i: (i, 0)),
        out_shape=jax.ShapeDtypeStruct((N_NODES, D), jnp.float32),
    )(parts, b.reshape(1, D))


def kernel(x, edge_index, edge_weight, W, b):
    h = _matmul(x, W)
    row = edge_index[0]
    col = edge_index[1]
    parts = _sc_spmm(h, row, col, edge_weight)
    return _combine(parts, b)
